# Initial kernel scaffold; baseline (speedup 1.0000x reference)
import jax
import jax.numpy as jnp
from jax.experimental import pallas as pl


def kernel(x, edge_index, edge_attr, W_node, b_node, W_l, b_l, W_r, gamma, beta, W_out, b_out):
    raise NotImplementedError("write your pallas kernel here")


# speedup vs baseline: 2.4772x; 2.4772x over previous
import jax
import jax.numpy as jnp
from jax import lax
from jax.experimental import pallas as pl
from jax.experimental.pallas import tpu as pltpu
from jax.experimental.pallas import tpu_sc as plsc

N = 10000
E = 320000
D = 128
NC = 2
NS = 16
NW = NC * NS
C = 128
KB = 8
K = -(-E // (NW * C * KB)) * KB
E_PAD = NW * K * C
ROWS_Z = 640
NP = NS * ROWS_Z

_mesh = plsc.VectorSubcoreMesh(core_axis_name="c", subcore_axis_name="s")


def _sc_agg_body(u, srci, dsti, zeros, out, acc, src_v, dst_v, rows_v, gsem):
    c = lax.axis_index("c")
    s = lax.axis_index("s")
    wid = s * NC + c

    pltpu.sync_copy(zeros, rows_v)
    for k in range(ROWS_Z // C):
        pltpu.sync_copy(rows_v, acc.at[pl.ds(s * ROWS_Z + k * C, C)])
    plsc.subcore_barrier()

    def block(ib, carry):
        pltpu.sync_copy(srci.at[wid, pl.ds(ib * KB, KB)], src_v)
        pltpu.sync_copy(dsti.at[wid, pl.ds(ib * KB, KB)], dst_v)

        def step(j, c2):
            pltpu.async_copy(u.at[src_v.at[j]], rows_v, gsem).wait()
            pltpu.sync_copy(rows_v, acc.at[dst_v.at[j]], add=True)
            return c2

        return lax.fori_loop(0, KB, step, carry)

    lax.fori_loop(0, K // KB, block, 0)
    plsc.subcore_barrier()

    ob = c * NP + s * ROWS_Z
    ab = s * ROWS_Z
    for k in range(ROWS_Z // C):
        pltpu.sync_copy(acc.at[pl.ds(ab + k * C, C)], rows_v)
        pltpu.sync_copy(rows_v, out.at[pl.ds(ob + k * C, C)])


_sc_agg = pl.kernel(
    _sc_agg_body,
    out_type=jax.ShapeDtypeStruct((NC * NP, D), jnp.float32),
    mesh=_mesh,
    scratch_types=[
        pltpu.VMEM_SHARED((NP, D), jnp.float32),
        pltpu.VMEM((KB, C), jnp.int32),
        pltpu.VMEM((KB, C), jnp.int32),
        pltpu.VMEM((C, D), jnp.float32),
        pltpu.SemaphoreType.DMA,
    ],
)


_R = 1024
_GRID = NP // _R


def _row_spec(r, w):
    return pl.BlockSpec((r, w), lambda i: (i, 0))


def _full_spec(shape):
    return pl.BlockSpec(shape, lambda i: (0,) * len(shape))


def _ln_relu(h, gamma, beta):
    m = jnp.mean(h, axis=-1, keepdims=True)
    v = jnp.mean((h - m) ** 2, axis=-1, keepdims=True)
    z = (h - m) * lax.rsqrt(v + 1e-5) * gamma + beta
    return jnp.maximum(z, 0.0)


def _cnt_col(ca, cb):
    return jnp.maximum(ca + cb, 1.0)


def _tc1_body(x, WnT, bn, WlT, bl, WrT, u1, r1):
    h = jnp.dot(x[...], WnT[...], preferred_element_type=jnp.float32) + bn[...]
    u1[...] = jnp.dot(h, WlT[...], preferred_element_type=jnp.float32)
    r1[...] = jnp.dot(h, WrT[...], preferred_element_type=jnp.float32) + bl[...]


def _tc2_body(sa, sb, ca, cb, r1, gamma, beta, WlT, bl, WrT, u2, r2, h1_out):
    h1 = (sa[...] + sb[...]) / _cnt_col(ca[...], cb[...]) + r1[...]
    h1_out[...] = h1
    z = _ln_relu(h1, gamma[...], beta[...])
    u2[...] = jnp.dot(z, WlT[...], preferred_element_type=jnp.float32)
    r2[...] = jnp.dot(z, WrT[...], preferred_element_type=jnp.float32) + bl[...]


def _tc3_body(sa, sb, ca, cb, r2, h1, gamma, beta, WoT, bo, y):
    z2 = (sa[...] + sb[...]) / _cnt_col(ca[...], cb[...]) + r2[...]
    h3 = z2 + h1[...]
    hf = _ln_relu(h3, gamma[...], beta[...])
    y[...] = jnp.dot(hf, WoT[...], preferred_element_type=jnp.float32) + bo[...]


def kernel(x, edge_index, edge_attr, W_node, b_node, W_l, b_l, W_r,
           gamma, beta, W_out, b_out):
    del edge_attr
    src = edge_index[0].astype(jnp.int32)
    dst = edge_index[1].astype(jnp.int32)
    pad = E_PAD - E
    srci = jnp.concatenate([src, jnp.zeros((pad,), jnp.int32)]).reshape(NW, K, C)
    dsti = jnp.concatenate([dst, jnp.full((pad,), N, jnp.int32)]).reshape(NW, K, C)
    zeros = jnp.zeros((C, D), jnp.float32)
    xp = jnp.pad(x, ((0, NP - N), (0, 0)))

    WnT = W_node.T
    WlT = W_l.T
    WrT = W_r.T
    WoT = W_out.T
    bn = b_node.reshape(1, D)
    bl = b_l.reshape(1, D)
    bo = b_out.reshape(1, -1)
    g2 = gamma.reshape(1, D)
    be = beta.reshape(1, D)

    w128 = _full_spec((D, D))
    v128 = _full_spec((1, D))
    cspec = _row_spec(_R, 1)

    u1, r1 = pl.pallas_call(
        _tc1_body,
        grid=(_GRID,),
        in_specs=[_row_spec(_R, D), w128, v128, w128, v128, w128],
        out_specs=[_row_spec(_R, D), _row_spec(_R, D)],
        out_shape=[jax.ShapeDtypeStruct((NP, D), jnp.float32)] * 2,
    )(xp, WnT, bn, WlT, bl, WrT)

    ones_t = jnp.ones((NP, D), jnp.float32)
    cnt_s = _sc_agg(ones_t, srci, dsti, zeros)
    ca = cnt_s[:NP, 0:1]
    cb = cnt_s[NP:, 0:1]
    s1 = _sc_agg(u1, srci, dsti, zeros)

    u2, r2, h1 = pl.pallas_call(
        _tc2_body,
        grid=(_GRID,),
        in_specs=[_row_spec(_R, D), _row_spec(_R, D), cspec, cspec,
                  _row_spec(_R, D), v128, v128, w128, v128, w128],
        out_specs=[_row_spec(_R, D)] * 3,
        out_shape=[jax.ShapeDtypeStruct((NP, D), jnp.float32)] * 3,
    )(s1[:NP], s1[NP:], ca, cb, r1, g2, be, WlT, bl, WrT)

    s2 = _sc_agg(u2, srci, dsti, zeros)

    y = pl.pallas_call(
        _tc3_body,
        grid=(_GRID,),
        in_specs=[_row_spec(_R, D), _row_spec(_R, D), cspec, cspec,
                  _row_spec(_R, D), _row_spec(_R, D),
                  v128, v128, w128, v128],
        out_specs=_row_spec(_R, D),
        out_shape=jax.ShapeDtypeStruct((NP, D), jnp.float32),
    )(s2[:NP], s2[NP:], ca, cb, r2, h1, g2, be, WoT, bo)

    return y[:N]
